# baseline (speedup 1.0000x reference)
import jax
import jax.numpy as jnp
from jax.experimental import pallas as pl
from jax.experimental.pallas import tpu as pltpu

_B = 16
_MAX_T = 4096
_S_DIM = 64
_HIDDEN = 128
_NA = 4
_BT = 512
_NT = _MAX_T // _BT


def _traj_kernel(lens_ref, s_ref, a_ref, w1_ref, b1_ref, w2_ref, b2_ref,
                 out_ref):
    i = pl.program_id(0)
    tb = pl.program_id(1)

    @pl.when(jnp.logical_and(i == 0, tb == 0))
    def _init():
        out_ref[0, 0] = 0.0

    length = lens_ref[i]

    @pl.when(tb * _BT < length)
    def _body():
        x = s_ref[0]
        h = jnp.tanh(
            jax.lax.dot_general(x, w1_ref[...], (((1,), (0,)), ((), ())),
                                preferred_element_type=jnp.float32)
            + b1_ref[0])
        z = jax.lax.dot_general(h, w2_ref[...], (((1,), (0,)), ((), ())),
                                preferred_element_type=jnp.float32)
        z = z + b2_ref[0]
        m = jnp.max(z, axis=-1, keepdims=True)
        lse = m + jnp.log(jnp.sum(jnp.exp(z - m), axis=-1, keepdims=True))
        logp = z - lse
        onehot = (a_ref[0] ==
                  jax.lax.broadcasted_iota(jnp.int32, (_BT, _NA), 1))
        t = tb * _BT + jax.lax.broadcasted_iota(jnp.int32, (_BT, _NA), 0)
        sel = jnp.where(onehot & (t < length), logp, 0.0)
        out_ref[0, 0] += jnp.sum(sel)


def _clamped_tb(tb, lens, i):
    return jnp.minimum(tb, (lens[i] - 1) // _BT)


def kernel(s_i_batch, actions_batch, lengths, W1, b1, W2, b2):
    w2_4 = W2[:, :_NA]
    b2_4 = b2[:_NA].reshape(1, _NA)
    b1_2 = b1.reshape(1, _HIDDEN)
    actions3 = actions_batch[..., None]

    grid_spec = pltpu.PrefetchScalarGridSpec(
        num_scalar_prefetch=1,
        grid=(_B, _NT),
        in_specs=[
            pl.BlockSpec((1, _BT, _S_DIM),
                         lambda i, tb, lens: (i, _clamped_tb(tb, lens, i), 0)),
            pl.BlockSpec((1, _BT, 1),
                         lambda i, tb, lens: (i, _clamped_tb(tb, lens, i), 0)),
            pl.BlockSpec((_S_DIM, _HIDDEN), lambda i, tb, lens: (0, 0)),
            pl.BlockSpec((1, _HIDDEN), lambda i, tb, lens: (0, 0)),
            pl.BlockSpec((_HIDDEN, _NA), lambda i, tb, lens: (0, 0)),
            pl.BlockSpec((1, _NA), lambda i, tb, lens: (0, 0)),
        ],
        out_specs=pl.BlockSpec(memory_space=pltpu.SMEM),
    )

    total = pl.pallas_call(
        _traj_kernel,
        grid_spec=grid_spec,
        out_shape=jax.ShapeDtypeStruct((1, 1), jnp.float32),
        compiler_params=pltpu.CompilerParams(
            dimension_semantics=("arbitrary", "arbitrary")),
    )(lengths, s_i_batch, actions3, W1, b1_2, w2_4, b2_4)
    return -total[0, 0]


# speedup vs baseline: 1.2179x; 1.2179x over previous
import jax
import jax.numpy as jnp
from jax.experimental import pallas as pl
from jax.experimental.pallas import tpu as pltpu

_B = 16
_MAX_T = 4096
_S_DIM = 64
_HIDDEN = 128
_NA = 4
_BT = 1024
_NT = _MAX_T // _BT


def _traj_kernel(lens_ref, s_ref, a_ref, w1_ref, b1_ref, w2_ref, b2_ref,
                 out_ref):
    i = pl.program_id(0)
    tb = pl.program_id(1)

    @pl.when(tb == 0)
    def _init():
        out_ref[0, 0, 0] = 0.0

    length = lens_ref[i]

    @pl.when(tb * _BT < length)
    def _body():
        x = s_ref[0]
        h = jnp.tanh(
            jax.lax.dot_general(x, w1_ref[...], (((1,), (0,)), ((), ())),
                                preferred_element_type=jnp.float32)
            + b1_ref[0])
        z = jax.lax.dot_general(h, w2_ref[...], (((1,), (0,)), ((), ())),
                                preferred_element_type=jnp.float32)
        z = z + b2_ref[0]
        m = jnp.max(z, axis=-1, keepdims=True)
        lse = m + jnp.log(jnp.sum(jnp.exp(z - m), axis=-1, keepdims=True))
        logp = z - lse
        onehot = (a_ref[0] ==
                  jax.lax.broadcasted_iota(jnp.int32, (_BT, _NA), 1))
        t = tb * _BT + jax.lax.broadcasted_iota(jnp.int32, (_BT, _NA), 0)
        sel = jnp.where(onehot & (t < length), logp, 0.0)
        out_ref[0, 0, 0] += jnp.sum(sel)


def _clamped_tb(tb, lens, i):
    return jnp.minimum(tb, (lens[i] - 1) // _BT)


def kernel(s_i_batch, actions_batch, lengths, W1, b1, W2, b2):
    w2_4 = W2[:, :_NA]
    b2_4 = b2[:_NA].reshape(1, _NA)
    b1_2 = b1.reshape(1, _HIDDEN)
    actions3 = actions_batch[..., None]

    grid_spec = pltpu.PrefetchScalarGridSpec(
        num_scalar_prefetch=1,
        grid=(_B, _NT),
        in_specs=[
            pl.BlockSpec((1, _BT, _S_DIM),
                         lambda i, tb, lens: (i, _clamped_tb(tb, lens, i), 0)),
            pl.BlockSpec((1, _BT, 1),
                         lambda i, tb, lens: (i, _clamped_tb(tb, lens, i), 0)),
            pl.BlockSpec((_S_DIM, _HIDDEN), lambda i, tb, lens: (0, 0)),
            pl.BlockSpec((1, _HIDDEN), lambda i, tb, lens: (0, 0)),
            pl.BlockSpec((_HIDDEN, _NA), lambda i, tb, lens: (0, 0)),
            pl.BlockSpec((1, _NA), lambda i, tb, lens: (0, 0)),
        ],
        out_specs=pl.BlockSpec((1, 1, 1), lambda i, tb, lens: (i, 0, 0),
                               memory_space=pltpu.SMEM),
    )

    partials = pl.pallas_call(
        _traj_kernel,
        grid_spec=grid_spec,
        out_shape=jax.ShapeDtypeStruct((_B, 1, 1), jnp.float32),
        compiler_params=pltpu.CompilerParams(
            dimension_semantics=("parallel", "arbitrary")),
    )(lengths, s_i_batch, actions3, W1, b1_2, w2_4, b2_4)
    return -jnp.sum(partials)


# speedup vs baseline: 2.3260x; 1.9099x over previous
import jax
import jax.numpy as jnp
from jax.experimental import pallas as pl
from jax.experimental.pallas import tpu as pltpu

_B = 16
_MAX_T = 4096
_S_DIM = 64
_HIDDEN = 128
_NA = 4


def _traj_kernel(lens_ref, s_ref, a_ref, w1_ref, b1_ref, w2t_ref, b2_ref,
                 out_ref):
    i = pl.program_id(0)
    length = lens_ref[i]

    x = s_ref[0]
    h = jnp.tanh(
        jax.lax.dot_general(x, w1_ref[...], (((1,), (0,)), ((), ())),
                            preferred_element_type=jnp.float32)
        + b1_ref[0])
    zt = jax.lax.dot_general(w2t_ref[...], h, (((1,), (1,)), ((), ())),
                             preferred_element_type=jnp.float32)
    zt = zt + b2_ref[...]
    m = jnp.max(zt, axis=0, keepdims=True)
    lse = m + jnp.log(jnp.sum(jnp.exp(zt - m), axis=0, keepdims=True))
    logp = zt - lse
    onehot = (a_ref[0] ==
              jax.lax.broadcasted_iota(jnp.int32, (_NA, _MAX_T), 0))
    t = jax.lax.broadcasted_iota(jnp.int32, (1, _MAX_T), 1)
    sel = jnp.where(onehot & (t < length), logp, 0.0)
    out_ref[0, 0, 0] = jnp.sum(sel)


def kernel(s_i_batch, actions_batch, lengths, W1, b1, W2, b2):
    w2t = W2[:, :_NA].T
    b1r = b1.reshape(1, _HIDDEN)
    b2c = b2[:_NA].reshape(_NA, 1)
    actions3 = actions_batch.reshape(_B, 1, _MAX_T)

    grid_spec = pltpu.PrefetchScalarGridSpec(
        num_scalar_prefetch=1,
        grid=(_B,),
        in_specs=[
            pl.BlockSpec((1, _MAX_T, _S_DIM), lambda i, lens: (i, 0, 0)),
            pl.BlockSpec((1, 1, _MAX_T), lambda i, lens: (i, 0, 0)),
            pl.BlockSpec((_S_DIM, _HIDDEN), lambda i, lens: (0, 0)),
            pl.BlockSpec((1, _HIDDEN), lambda i, lens: (0, 0)),
            pl.BlockSpec((_NA, _HIDDEN), lambda i, lens: (0, 0)),
            pl.BlockSpec((_NA, 1), lambda i, lens: (0, 0)),
        ],
        out_specs=pl.BlockSpec((1, 1, 1), lambda i, lens: (i, 0, 0),
                               memory_space=pltpu.SMEM),
    )

    partials = pl.pallas_call(
        _traj_kernel,
        grid_spec=grid_spec,
        out_shape=jax.ShapeDtypeStruct((_B, 1, 1), jnp.float32),
        compiler_params=pltpu.CompilerParams(
            dimension_semantics=("arbitrary",)),
    )(lengths, s_i_batch, actions3, W1, b1r, w2t, b2c)
    return -jnp.sum(partials)
